# baseline (speedup 1.0000x reference)
import functools

import jax
import jax.numpy as jnp
import numpy as np
from jax import lax
from jax.experimental import pallas as pl
from jax.experimental.pallas import tpu as pltpu
from jax.experimental.pallas import tpu_sc as plsc

_NUM_EMB = 8192
_EMB_DIM = 32
_BETA = 0.25

_RBLK = 256
_KBLK = 512
_NR = _NUM_EMB // _RBLK
_NK = _NUM_EMB // _KBLK
_STRIP = 2048
_BLKS_PER_STRIP = _STRIP // _KBLK
_BIG_I32 = 2**30


def _rnd_bf16(x):
    u = lax.bitcast_convert_type(x, jnp.uint32)
    u = (u + np.uint32(0x7FFF) + ((u >> 16) & np.uint32(1))) & np.uint32(0xFFFF0000)
    return lax.bitcast_convert_type(u, jnp.float32)


def _argmin_body(f_ref, cb_ref, a_ref, cn_ref, idx_ref, loss_ref,
                 sv_ref, si_ref, av_ref, ai_ref, ad_ref, acc_ref):
    r = pl.program_id(0)
    k = pl.program_id(1)
    lhs = _rnd_bf16(2.0 * f_ref[...])
    rhs = _rnd_bf16(cb_ref[...])
    mm = lax.dot_general(
        lhs, rhs,
        dimension_numbers=(((1,), (1,)), ((), ())),
        preferred_element_type=jnp.float32,
    )
    d = (a_ref[...] - mm) + cn_ref[...]
    lmin = jnp.min(d, axis=1, keepdims=True)
    kidx = lax.broadcasted_iota(jnp.int32, d.shape, 1) + k * _KBLK
    lidx = jnp.min(jnp.where(d == lmin, kidx, _BIG_I32), axis=1, keepdims=True)

    @pl.when(k % _BLKS_PER_STRIP == 0)
    def _():
        sv_ref[...] = lmin
        si_ref[...] = lidx

    @pl.when(k % _BLKS_PER_STRIP != 0)
    def _():
        better = lmin < sv_ref[...]
        si_ref[...] = jnp.where(better, lidx, si_ref[...])
        sv_ref[...] = jnp.where(better, lmin, sv_ref[...])

    @pl.when(k == _BLKS_PER_STRIP - 1)
    def _():
        av_ref[...] = _rnd_bf16(sv_ref[...])
        ai_ref[...] = si_ref[...]
        ad_ref[...] = sv_ref[...]

    @pl.when((k % _BLKS_PER_STRIP == _BLKS_PER_STRIP - 1)
             & (k > _BLKS_PER_STRIP - 1))
    def _():
        steal = sv_ref[...] < av_ref[...]
        av_ref[...] = jnp.where(steal, _rnd_bf16(sv_ref[...]), av_ref[...])
        ai_ref[...] = jnp.where(steal, si_ref[...], ai_ref[...])
        ad_ref[...] = jnp.where(steal, sv_ref[...], ad_ref[...])

    @pl.when(k == _NK - 1)
    def _():
        idx_ref[...] = ai_ref[...]
        s = jnp.sum(ad_ref[...])

        @pl.when(r == 0)
        def _():
            acc_ref[0] = s

        @pl.when(r > 0)
        def _():
            acc_ref[0] = acc_ref[0] + s

        @pl.when(r == _NR - 1)
        def _():
            loss_ref[...] = jnp.full(
                (1, 1), acc_ref[0] * ((1.0 + _BETA) / (8192.0 * 32.0)),
                dtype=jnp.float32)


def _argmin_call(flat, codebook, a, cn):
    return pl.pallas_call(
        _argmin_body,
        grid=(_NR, _NK),
        in_specs=[
            pl.BlockSpec((_RBLK, _EMB_DIM), lambda r, k: (r, 0)),
            pl.BlockSpec((_KBLK, _EMB_DIM), lambda r, k: (k, 0)),
            pl.BlockSpec((_RBLK, 1), lambda r, k: (r, 0)),
            pl.BlockSpec((1, _KBLK), lambda r, k: (0, k)),
        ],
        out_specs=[
            pl.BlockSpec((_RBLK, 1), lambda r, k: (r, 0)),
            pl.BlockSpec((1, 1), lambda r, k: (0, 0)),
        ],
        out_shape=[
            jax.ShapeDtypeStruct((8192, 1), jnp.int32),
            jax.ShapeDtypeStruct((1, 1), jnp.float32),
        ],
        scratch_shapes=[
            pltpu.VMEM((_RBLK, 1), jnp.float32),
            pltpu.VMEM((_RBLK, 1), jnp.int32),
            pltpu.VMEM((_RBLK, 1), jnp.float32),
            pltpu.VMEM((_RBLK, 1), jnp.int32),
            pltpu.VMEM((_RBLK, 1), jnp.float32),
            pltpu.SMEM((1,), jnp.float32),
        ],
        compiler_params=pltpu.CompilerParams(
            dimension_semantics=("arbitrary", "arbitrary"),
        ),
    )(flat, codebook, a, cn)


_SC_MESH = plsc.VectorSubcoreMesh(core_axis_name="c", subcore_axis_name="s")
_ROWS_PER_W = 2
_PAD_DIM = 128


def _gather_body(cb_hbm, idx_hbm, out_hbm, idx_v, rows_v, sem):
    wid = lax.axis_index("s") * 2 + lax.axis_index("c")
    base = wid * _ROWS_PER_W
    pltpu.sync_copy(idx_hbm.at[pl.ds(base, _ROWS_PER_W)], idx_v)
    c0 = pltpu.async_copy(cb_hbm.at[idx_v.at[0]], rows_v.at[0], sem)
    c1 = pltpu.async_copy(cb_hbm.at[idx_v.at[1]], rows_v.at[1], sem)
    c0.wait()
    c1.wait()
    pltpu.sync_copy(rows_v, out_hbm.at[pl.ds(base, _ROWS_PER_W)])


_gather_call = functools.partial(
    pl.kernel,
    mesh=_SC_MESH,
    out_type=jax.ShapeDtypeStruct((64, 128, _PAD_DIM), jnp.float32),
    scratch_types=[
        pltpu.VMEM((_ROWS_PER_W, 128), jnp.int32),
        pltpu.VMEM((_ROWS_PER_W, 128, _PAD_DIM), jnp.float32),
        pltpu.SemaphoreType.DMA,
    ],
)(_gather_body)


def kernel(z, codebook):
    flat = z.reshape(-1, _EMB_DIM)
    a = jnp.sum(z ** 2, axis=2).reshape(-1, 1)
    cn = jnp.sum(codebook ** 2, axis=1)[None, :]
    idx, loss = _argmin_call(flat, codebook, a, cn)
    cb_pad = jnp.pad(codebook, ((0, 0), (0, _PAD_DIM - _EMB_DIM)))
    rows = _gather_call(cb_pad, idx.reshape(64, 128))
    zq = rows.reshape(8192, _PAD_DIM)[:, :_EMB_DIM].reshape(z.shape)
    quantized = z + (zq - z)
    return quantized, loss[0, 0]
